# baseline (speedup 1.0000x reference)
import functools

import jax
import jax.numpy as jnp
from jax import lax
from jax.experimental import pallas as pl
from jax.experimental.pallas import tpu as pltpu
from jax.experimental.pallas import tpu_sc as plsc

SCALE = 10.0
NC, NS, LANES = 2, 16, 16
NW = NC * NS

WT = 256
WO = 256


def _pack_bf16_pair(lo_f32, hi_f32):
  def rnd(v):
    u = lax.bitcast_convert_type(v, jnp.uint32)
    return (u + jnp.uint32(0x7FFF) + ((u >> jnp.uint32(16)) & jnp.uint32(1))
            ) >> jnp.uint32(16)
  w = rnd(lo_f32) | (rnd(hi_f32) << jnp.uint32(16))
  return lax.bitcast_convert_type(w, jnp.int32)


def _unpack_f32(w):
  lo = lax.bitcast_convert_type(w << jnp.int32(16), jnp.float32)
  hi = lax.bitcast_convert_type(w & jnp.int32(-65536), jnp.float32)
  return lo, hi


def _tables_body(h_ref, wx1a, wx1b, we1a, we1b, ta_ref, tb_ref):
  hb = h_ref[0]
  ta_ref[0] = _pack_bf16_pair(
      jnp.dot(hb, wx1a[...], preferred_element_type=jnp.float32),
      jnp.dot(hb, we1a[...], preferred_element_type=jnp.float32))
  tb_ref[0] = _pack_bf16_pair(
      jnp.dot(hb, wx1b[...], preferred_element_type=jnp.float32),
      jnp.dot(hb, we1b[...], preferred_element_type=jnp.float32))


def _make_tables(h, wx1a, wx1b, we1a, we1b):
  B, N, Dh = h.shape
  wspec = lambda s: pl.BlockSpec(s, lambda b: (0,) * len(s))
  return pl.pallas_call(
      _tables_body,
      grid=(B,),
      in_specs=[
          pl.BlockSpec((1, N, Dh), lambda b: (b, 0, 0)),
          wspec((Dh, 256)), wspec((Dh, 256)), wspec((Dh, 256)), wspec((Dh, 256)),
      ],
      out_specs=[
          pl.BlockSpec((1, N, WT), lambda b: (b, 0, 0)),
          pl.BlockSpec((1, N, WT), lambda b: (b, 0, 0)),
      ],
      out_shape=[
          jax.ShapeDtypeStruct((B, N, WT), jnp.int32),
          jax.ShapeDtypeStruct((B, N, WT), jnp.int32),
      ],
  )(h, wx1a, wx1b, we1a, we1b)


def _sc_gather(ta, tb, idx_i, idx_j):
  B, N, W = ta.shape
  E = idx_i.shape[1]
  epw = E // NW
  CH = 128
  nch = epw // CH
  mesh = plsc.VectorSubcoreMesh(core_axis_name="c", subcore_axis_name="s")

  @functools.partial(
      pl.kernel,
      mesh=mesh,
      out_type=[jax.ShapeDtypeStruct((B, E, W), jnp.int32),
                jax.ShapeDtypeStruct((B, E, W), jnp.int32)],
      scratch_types=[
          pltpu.VMEM((epw,), jnp.int32),
          pltpu.VMEM((epw,), jnp.int32),
          pltpu.VMEM((CH, W), jnp.int32),
          pltpu.VMEM((CH, W), jnp.int32),
          pltpu.SemaphoreType.DMA,
          pltpu.SemaphoreType.DMA,
          pltpu.SemaphoreType.DMA,
          pltpu.SemaphoreType.DMA,
          pltpu.SemaphoreType.DMA,
          pltpu.SemaphoreType.DMA,
      ],
  )
  def k(ta_h, tb_h, ii_h, ij_h, ga_h, gb_h,
        idx0, idx1, rows0, rows1, is0, is1, gs0, gs1, ws0, ws1):
    wid = lax.axis_index("s") * NC + lax.axis_index("c")
    base = wid * epw
    ivs = (idx0, idx1)
    isem = (is0, is1)
    bufs = (rows0, rows1)
    gsem = (gs0, gs1)
    wsem = (ws0, ws1)

    seqs = [(b, tab, idx_hbm, out_hbm)
            for b in range(B)
            for tab, idx_hbm, out_hbm in ((ta_h, ii_h, ga_h),
                                          (tb_h, ij_h, gb_h))]
    S = len(seqs)
    T = S * nch
    ih, gh, wh = {}, {}, {}

    b0, _, idx_hbm0, _ = seqs[0]
    ih[0] = pltpu.async_copy(idx_hbm0.at[b0, pl.ds(base, epw)], ivs[0], isem[0])

    for t in range(T):
      s, c, k2 = t // nch, t % nch, t & 1
      b, tab, idx_hbm, out_hbm = seqs[s]
      if c == 0:
        ih[s].wait()
      if t >= 2:
        wh[t - 2].wait()
      gh[t] = pltpu.async_copy(
          tab.at[b].at[ivs[s & 1].at[pl.ds(c * CH, CH)]], bufs[k2], gsem[k2])
      if t >= 1:
        pc, pk = t - 1, (t - 1) & 1
        ps, pcc = pc // nch, pc % nch
        pb, _, _, pout = seqs[ps]
        gh[pc].wait()
        wh[pc] = pltpu.async_copy(
            bufs[pk], pout.at[pb, pl.ds(base + pcc * CH, CH)], wsem[pk])
      if c == 0 and s + 1 < S:
        nb, _, nidx, _ = seqs[s + 1]
        ih[s + 1] = pltpu.async_copy(
            nidx.at[nb, pl.ds(base, epw)], ivs[(s + 1) & 1], isem[(s + 1) & 1])

    lt, lk = T - 1, (T - 1) & 1
    lb, _, _, lout = seqs[S - 1]
    gh[lt].wait()
    wh[lt] = pltpu.async_copy(
        bufs[lk], lout.at[lb, pl.ds(base + (nch - 1) * CH, CH)], wsem[lk])
    wh[lt - 1].wait()
    wh[lt].wait()

  return k(ta, tb, idx_i, idx_j)


def _silu(v):
  return v * jax.nn.sigmoid(v)


def _edge_body(ga_ref, gb_ref, ea_ref, em_ref, x_ref, ii_ref, ij_ref,
               wx1ea, wx1d, bx1, wx2, bx2, wx3t,
               we1ea, we1d, be1, we2, be2, wat, ba, eacc_ref, xacc_ref):
  gxa, gea = _unpack_f32(ga_ref[0])
  gxb, geb = _unpack_f32(gb_ref[0])
  gx = gxa + gxb
  ge = gea + geb
  ea = ea_ref[0]
  msk = em_ref[0]
  xb = x_ref[0]
  ii = ii_ref[0, 0]
  ij = ij_ref[0, 0]
  eb = gx.shape[0]
  n = xb.shape[0]

  lanes = lax.broadcasted_iota(jnp.int32, (eb, n), 1)
  oh_i = (ii[:, None] == lanes).astype(jnp.float32)
  oh_j = (ij[:, None] == lanes).astype(jnp.float32)
  oh_i_b = oh_i.astype(jnp.bfloat16)
  x_i = jnp.dot(oh_i, xb, preferred_element_type=jnp.float32)
  x_j = jnp.dot(oh_j, xb, preferred_element_type=jnp.float32)

  diff = (x_i - x_j) * msk
  d2 = jnp.sum(diff * diff, axis=-1, keepdims=True)
  d = jnp.sqrt(d2)

  ea_x = jnp.dot(ea, wx1ea[...], preferred_element_type=jnp.float32)
  t1x = msk * (gx + ea_x) + (msk * d2) * wx1d[...] + bx1[...]
  u = _silu(t1x)
  w2 = _silu(jnp.dot(u.astype(jnp.bfloat16), wx2[...].astype(jnp.bfloat16),
                     preferred_element_type=jnp.float32) + bx2[...])
  s = jnp.tanh(jnp.sum(w2 * wx3t[...], axis=-1, keepdims=True))
  xm = diff / (d + 1.0) * s * SCALE

  ea_e = jnp.dot(ea, we1ea[...], preferred_element_type=jnp.float32)
  t1e = msk * (ge + ea_e) + (msk * d2) * we1d[...] + be1[...]
  m1 = _silu(jnp.dot(_silu(t1e).astype(jnp.bfloat16),
                     we2[...].astype(jnp.bfloat16),
                     preferred_element_type=jnp.float32) + be2[...])
  att = jax.nn.sigmoid(jnp.sum(m1 * wat[...], axis=-1, keepdims=True) + ba[...])

  xpart = lax.dot_general(oh_i, xm, (((0,), (0,)), ((), ())),
                          preferred_element_type=jnp.float32)
  epart = lax.dot_general(oh_i_b, (att * m1).astype(jnp.bfloat16),
                          (((0,), (0,)), ((), ())),
                          preferred_element_type=jnp.float32)

  @pl.when(pl.program_id(1) == 0)
  def _init():
    xacc_ref[0] = jnp.zeros_like(xacc_ref[0])
    eacc_ref[0] = jnp.zeros_like(eacc_ref[0])

  xacc_ref[0] += xpart
  eacc_ref[0] += epart


def _edge_mlp(ga, gb, edge_attr, edge_mask, x, ii3, ij3,
              wx1ea, wx1d, bx1, wx2, bx2, wx3t,
              we1ea, we1d, be1, we2, be2, wat, ba):
  B, E, _ = ga.shape
  N = x.shape[1]
  EBLK = 2048
  wspec = lambda s: pl.BlockSpec(s, lambda b, e: (0,) * len(s))
  return pl.pallas_call(
      _edge_body,
      grid=(B, E // EBLK),
      in_specs=[
          pl.BlockSpec((1, EBLK, WT), lambda b, e: (b, e, 0)),
          pl.BlockSpec((1, EBLK, WT), lambda b, e: (b, e, 0)),
          pl.BlockSpec((1, EBLK, 16), lambda b, e: (b, e, 0)),
          pl.BlockSpec((1, EBLK, 1), lambda b, e: (b, e, 0)),
          pl.BlockSpec((1, N, 3), lambda b, e: (b, 0, 0)),
          pl.BlockSpec((1, 1, EBLK), lambda b, e: (b * (E // EBLK) + e, 0, 0)),
          pl.BlockSpec((1, 1, EBLK), lambda b, e: (b * (E // EBLK) + e, 0, 0)),
          wspec((16, 256)), wspec((1, 256)), wspec((1, 256)),
          wspec((256, 256)), wspec((1, 256)), wspec((1, 256)),
          wspec((16, 256)), wspec((1, 256)), wspec((1, 256)),
          wspec((256, 256)), wspec((1, 256)), wspec((1, 256)), wspec((1, 1)),
      ],
      out_specs=[
          pl.BlockSpec((1, N, WO), lambda b, e: (b, 0, 0)),
          pl.BlockSpec((1, N, 3), lambda b, e: (b, 0, 0)),
      ],
      out_shape=[
          jax.ShapeDtypeStruct((B, N, WO), jnp.float32),
          jax.ShapeDtypeStruct((B, N, 3), jnp.float32),
      ],
  )(ga, gb, edge_attr, edge_mask, x, ii3, ij3,
    wx1ea, wx1d, bx1, wx2, bx2, wx3t,
    we1ea, we1d, be1, we2, be2, wat, ba)


def _node_body(x_ref, h_ref, acc_ref, xacc_ref, nm_ref, wh1h, wh1e, bh1,
               wh2, bh2, xo_ref, ho_ref):
  xb = x_ref[0]
  hb = h_ref[0]
  nm = nm_ref[0]
  em_agg = acc_ref[0]
  xo_ref[0] = (xb + xacc_ref[0]) * nm
  t = _silu(jnp.dot(hb, wh1h[...], preferred_element_type=jnp.float32)
            + jnp.dot(em_agg, wh1e[...], preferred_element_type=jnp.float32)
            + bh1[...])
  ho_ref[0] = (hb + jnp.dot(t, wh2[...], preferred_element_type=jnp.float32)
               + bh2[...]) * nm


def _node_update(x, h, acc, xacc, node_mask, wh1h, wh1e, bh1, wh2, bh2):
  B, N, Dh = h.shape
  wspec = lambda s: pl.BlockSpec(s, lambda b: (0,) * len(s))
  return pl.pallas_call(
      _node_body,
      grid=(B,),
      in_specs=[
          pl.BlockSpec((1, N, 3), lambda b: (b, 0, 0)),
          pl.BlockSpec((1, N, Dh), lambda b: (b, 0, 0)),
          pl.BlockSpec((1, N, WO), lambda b: (b, 0, 0)),
          pl.BlockSpec((1, N, 3), lambda b: (b, 0, 0)),
          pl.BlockSpec((1, N, 1), lambda b: (b, 0, 0)),
          wspec((Dh, 256)), wspec((256, 256)), wspec((1, 256)),
          wspec((256, Dh)), wspec((1, Dh)),
      ],
      out_specs=[
          pl.BlockSpec((1, N, 3), lambda b: (b, 0, 0)),
          pl.BlockSpec((1, N, Dh), lambda b: (b, 0, 0)),
      ],
      out_shape=[
          jax.ShapeDtypeStruct((B, N, 3), jnp.float32),
          jax.ShapeDtypeStruct((B, N, Dh), jnp.float32),
      ],
  )(x, h, acc, xacc, node_mask, wh1h, wh1e, bh1, wh2, bh2)


def kernel(x, h, edge_attr, edge_indices, node_mask, edge_mask,
           We1, be1, We2, be2, Wa, ba, Wh1, bh1, Wh2, bh2,
           Wx1, bx1, Wx2, bx2, Wx3):
  B, N, Dh = h.shape
  E = edge_attr.shape[1]
  EBLK = 2048

  idx_i = edge_indices[..., 0].astype(jnp.int32)
  idx_j = edge_indices[..., 1].astype(jnp.int32)
  ii3 = idx_i.reshape(B * (E // EBLK), 1, EBLK)
  ij3 = idx_j.reshape(B * (E // EBLK), 1, EBLK)

  ta, tb = _make_tables(h, Wx1[0:256], Wx1[256:512], We1[0:256], We1[256:512])
  ga, gb = _sc_gather(ta, tb, idx_i, idx_j)
  eacc, xacc = _edge_mlp(
      ga, gb, edge_attr, edge_mask, x, ii3, ij3,
      Wx1[513:529], Wx1[512:513], bx1.reshape(1, 256),
      Wx2, bx2.reshape(1, 256), Wx3.reshape(1, 256),
      We1[513:529], We1[512:513], be1.reshape(1, 256),
      We2, be2.reshape(1, 256), Wa.reshape(1, 256), ba.reshape(1, 1))
  return _node_update(x, h, eacc, xacc, node_mask,
                      Wh1[0:Dh], Wh1[Dh:Dh + 256], bh1.reshape(1, 256),
                      Wh2, bh2.reshape(1, Dh))


# speedup vs baseline: 1.0850x; 1.0850x over previous
import functools

import jax
import jax.numpy as jnp
from jax import lax
from jax.experimental import pallas as pl
from jax.experimental.pallas import tpu as pltpu
from jax.experimental.pallas import tpu_sc as plsc

SCALE = 10.0
NC, NS, LANES = 2, 16, 16
NW = NC * NS

WT = 256
WO = 256


def _pack_bf16_pair(lo_f32, hi_f32):
  def rnd(v):
    u = lax.bitcast_convert_type(v, jnp.uint32)
    return (u + jnp.uint32(0x7FFF) + ((u >> jnp.uint32(16)) & jnp.uint32(1))
            ) >> jnp.uint32(16)
  w = rnd(lo_f32) | (rnd(hi_f32) << jnp.uint32(16))
  return lax.bitcast_convert_type(w, jnp.int32)


def _unpack_f32(w):
  lo = lax.bitcast_convert_type(w << jnp.int32(16), jnp.float32)
  hi = lax.bitcast_convert_type(w & jnp.int32(-65536), jnp.float32)
  return lo, hi


def _tables_body(h_ref, wx1a, wx1b, we1a, we1b, ta_ref, tb_ref):
  hb = h_ref[0]
  ta_ref[0] = _pack_bf16_pair(
      jnp.dot(hb, wx1a[...], preferred_element_type=jnp.float32),
      jnp.dot(hb, we1a[...], preferred_element_type=jnp.float32))
  tb_ref[0] = _pack_bf16_pair(
      jnp.dot(hb, wx1b[...], preferred_element_type=jnp.float32),
      jnp.dot(hb, we1b[...], preferred_element_type=jnp.float32))


def _make_tables(h, wx1a, wx1b, we1a, we1b):
  B, N, Dh = h.shape
  wspec = lambda s: pl.BlockSpec(s, lambda b: (0,) * len(s))
  return pl.pallas_call(
      _tables_body,
      grid=(B,),
      in_specs=[
          pl.BlockSpec((1, N, Dh), lambda b: (b, 0, 0)),
          wspec((Dh, 256)), wspec((Dh, 256)), wspec((Dh, 256)), wspec((Dh, 256)),
      ],
      out_specs=[
          pl.BlockSpec((1, N, WT), lambda b: (b, 0, 0)),
          pl.BlockSpec((1, N, WT), lambda b: (b, 0, 0)),
      ],
      out_shape=[
          jax.ShapeDtypeStruct((B, N, WT), jnp.int32),
          jax.ShapeDtypeStruct((B, N, WT), jnp.int32),
      ],
  )(h, wx1a, wx1b, we1a, we1b)


def _sc_gather(ta, tb, idx_i, idx_j):
  B, N, W = ta.shape
  E = idx_i.shape[1]
  epw = E // NW
  CH = 128
  nch = epw // CH
  mesh = plsc.VectorSubcoreMesh(core_axis_name="c", subcore_axis_name="s")

  @functools.partial(
      pl.kernel,
      mesh=mesh,
      out_type=[jax.ShapeDtypeStruct((B, E, W), jnp.int32),
                jax.ShapeDtypeStruct((B, E, W), jnp.int32)],
      scratch_types=[
          pltpu.VMEM((epw,), jnp.int32),
          pltpu.VMEM((epw,), jnp.int32),
          pltpu.VMEM((CH, W), jnp.int32),
          pltpu.VMEM((CH, W), jnp.int32),
          pltpu.SemaphoreType.DMA,
          pltpu.SemaphoreType.DMA,
          pltpu.SemaphoreType.DMA,
          pltpu.SemaphoreType.DMA,
          pltpu.SemaphoreType.DMA,
          pltpu.SemaphoreType.DMA,
      ],
  )
  def k(ta_h, tb_h, ii_h, ij_h, ga_h, gb_h,
        idx0, idx1, rows0, rows1, is0, is1, gs0, gs1, ws0, ws1):
    wid = lax.axis_index("s") * NC + lax.axis_index("c")
    base = wid * epw
    ivs = (idx0, idx1)
    isem = (is0, is1)
    bufs = (rows0, rows1)
    gsem = (gs0, gs1)
    wsem = (ws0, ws1)

    seqs = [(b, tab, idx_hbm, out_hbm)
            for b in range(B)
            for tab, idx_hbm, out_hbm in ((ta_h, ii_h, ga_h),
                                          (tb_h, ij_h, gb_h))]
    S = len(seqs)
    T = S * nch
    ih, gh, wh = {}, {}, {}

    b0, _, idx_hbm0, _ = seqs[0]
    ih[0] = pltpu.async_copy(idx_hbm0.at[b0, pl.ds(base, epw)], ivs[0], isem[0])

    for t in range(T):
      s, c, k2 = t // nch, t % nch, t & 1
      b, tab, idx_hbm, out_hbm = seqs[s]
      if c == 0:
        ih[s].wait()
      if t >= 2:
        wh[t - 2].wait()
      gh[t] = pltpu.async_copy(
          tab.at[b].at[ivs[s & 1].at[pl.ds(c * CH, CH)]], bufs[k2], gsem[k2])
      if t >= 1:
        pc, pk = t - 1, (t - 1) & 1
        ps, pcc = pc // nch, pc % nch
        pb, _, _, pout = seqs[ps]
        gh[pc].wait()
        wh[pc] = pltpu.async_copy(
            bufs[pk], pout.at[pb, pl.ds(base + pcc * CH, CH)], wsem[pk])
      if c == 0 and s + 1 < S:
        nb, _, nidx, _ = seqs[s + 1]
        ih[s + 1] = pltpu.async_copy(
            nidx.at[nb, pl.ds(base, epw)], ivs[(s + 1) & 1], isem[(s + 1) & 1])

    lt, lk = T - 1, (T - 1) & 1
    lb, _, _, lout = seqs[S - 1]
    gh[lt].wait()
    wh[lt] = pltpu.async_copy(
        bufs[lk], lout.at[lb, pl.ds(base + (nch - 1) * CH, CH)], wsem[lk])
    wh[lt - 1].wait()
    wh[lt].wait()

  return k(ta, tb, idx_i, idx_j)


def _silu(v):
  return v * jax.nn.sigmoid(v)


def _edge_body(ga_ref, gb_ref, ea_ref, em_ref, x_ref, ii_ref, ij_ref,
               wx1ea, wx1d, bx1, wx2, bx2, wx3t,
               we1ea, we1d, be1, we2, be2, wat, ba, eacc_ref, xacc_ref):
  gxa, gea = _unpack_f32(ga_ref[0])
  gxb, geb = _unpack_f32(gb_ref[0])
  gx = gxa + gxb
  ge = gea + geb
  ea = ea_ref[0]
  msk = em_ref[0]
  xb = x_ref[0]
  ii = ii_ref[0, 0]
  ij = ij_ref[0, 0]
  eb = gx.shape[0]
  n = xb.shape[0]

  lanes = lax.broadcasted_iota(jnp.int32, (eb, n), 1)
  oh_i = (ii[:, None] == lanes).astype(jnp.float32)
  oh_j = (ij[:, None] == lanes).astype(jnp.float32)
  oh_i_b = oh_i.astype(jnp.bfloat16)
  x_i = jnp.dot(oh_i, xb, preferred_element_type=jnp.float32)
  x_j = jnp.dot(oh_j, xb, preferred_element_type=jnp.float32)

  diff = (x_i - x_j) * msk
  d2 = jnp.sum(diff * diff, axis=-1, keepdims=True)
  d = jnp.sqrt(d2)

  ea_x = jnp.dot(ea, wx1ea[...], preferred_element_type=jnp.float32)
  t1x = msk * (gx + ea_x) + (msk * d2) * wx1d[...] + bx1[...]
  u = _silu(t1x)
  w2 = _silu(jnp.dot(u.astype(jnp.bfloat16), wx2[...].astype(jnp.bfloat16),
                     preferred_element_type=jnp.float32) + bx2[...])
  s = jnp.tanh(jnp.sum(w2 * wx3t[...], axis=-1, keepdims=True))
  xm = diff / (d + 1.0) * s * SCALE

  ea_e = jnp.dot(ea, we1ea[...], preferred_element_type=jnp.float32)
  t1e = msk * (ge + ea_e) + (msk * d2) * we1d[...] + be1[...]
  m1 = _silu(jnp.dot(_silu(t1e).astype(jnp.bfloat16),
                     we2[...].astype(jnp.bfloat16),
                     preferred_element_type=jnp.float32) + be2[...])
  att = jax.nn.sigmoid(jnp.sum(m1 * wat[...], axis=-1, keepdims=True) + ba[...])

  xpart = lax.dot_general(oh_i, xm, (((0,), (0,)), ((), ())),
                          preferred_element_type=jnp.float32)
  epart = lax.dot_general(oh_i_b, (att * m1).astype(jnp.bfloat16),
                          (((0,), (0,)), ((), ())),
                          preferred_element_type=jnp.float32)

  @pl.when(pl.program_id(1) == 0)
  def _init():
    xacc_ref[0] = jnp.zeros_like(xacc_ref[0])
    eacc_ref[0] = jnp.zeros_like(eacc_ref[0])

  xacc_ref[0] += xpart
  eacc_ref[0] += epart


def _edge_mlp(ga, gb, edge_attr, edge_mask, x, ii3, ij3,
              wx1ea, wx1d, bx1, wx2, bx2, wx3t,
              we1ea, we1d, be1, we2, be2, wat, ba):
  B, E, _ = ga.shape
  N = x.shape[1]
  EBLK = 2048
  wspec = lambda s: pl.BlockSpec(s, lambda b, e: (0,) * len(s))
  return pl.pallas_call(
      _edge_body,
      grid=(B, E // EBLK),
      in_specs=[
          pl.BlockSpec((1, EBLK, WT), lambda b, e: (b, e, 0)),
          pl.BlockSpec((1, EBLK, WT), lambda b, e: (b, e, 0)),
          pl.BlockSpec((1, EBLK, 16), lambda b, e: (b, e, 0)),
          pl.BlockSpec((1, EBLK, 1), lambda b, e: (b, e, 0)),
          pl.BlockSpec((1, N, 3), lambda b, e: (b, 0, 0)),
          pl.BlockSpec((1, 1, EBLK), lambda b, e: (b * (E // EBLK) + e, 0, 0)),
          pl.BlockSpec((1, 1, EBLK), lambda b, e: (b * (E // EBLK) + e, 0, 0)),
          wspec((16, 256)), wspec((1, 256)), wspec((1, 256)),
          wspec((256, 256)), wspec((1, 256)), wspec((1, 256)),
          wspec((16, 256)), wspec((1, 256)), wspec((1, 256)),
          wspec((256, 256)), wspec((1, 256)), wspec((1, 256)), wspec((1, 1)),
      ],
      out_specs=[
          pl.BlockSpec((1, N, WO), lambda b, e: (b, 0, 0)),
          pl.BlockSpec((1, N, 3), lambda b, e: (b, 0, 0)),
      ],
      out_shape=[
          jax.ShapeDtypeStruct((B, N, WO), jnp.float32),
          jax.ShapeDtypeStruct((B, N, 3), jnp.float32),
      ],
  )(ga, gb, edge_attr, edge_mask, x, ii3, ij3,
    wx1ea, wx1d, bx1, wx2, bx2, wx3t,
    we1ea, we1d, be1, we2, be2, wat, ba)


def _node_body(x_ref, h_ref, acc_ref, xacc_ref, nm_ref, wh1h, wh1e, bh1,
               wh2, bh2, xo_ref, ho_ref):
  xb = x_ref[0]
  hb = h_ref[0]
  nm = nm_ref[0]
  em_agg = acc_ref[0]
  xo_ref[0] = (xb + xacc_ref[0]) * nm
  t = _silu(jnp.dot(hb, wh1h[...], preferred_element_type=jnp.float32)
            + jnp.dot(em_agg, wh1e[...], preferred_element_type=jnp.float32)
            + bh1[...])
  ho_ref[0] = (hb + jnp.dot(t, wh2[...], preferred_element_type=jnp.float32)
               + bh2[...]) * nm


def _node_update(x, h, acc, xacc, node_mask, wh1h, wh1e, bh1, wh2, bh2):
  B, N, Dh = h.shape
  wspec = lambda s: pl.BlockSpec(s, lambda b: (0,) * len(s))
  return pl.pallas_call(
      _node_body,
      grid=(B,),
      in_specs=[
          pl.BlockSpec((1, N, 3), lambda b: (b, 0, 0)),
          pl.BlockSpec((1, N, Dh), lambda b: (b, 0, 0)),
          pl.BlockSpec((1, N, WO), lambda b: (b, 0, 0)),
          pl.BlockSpec((1, N, 3), lambda b: (b, 0, 0)),
          pl.BlockSpec((1, N, 1), lambda b: (b, 0, 0)),
          wspec((Dh, 256)), wspec((256, 256)), wspec((1, 256)),
          wspec((256, Dh)), wspec((1, Dh)),
      ],
      out_specs=[
          pl.BlockSpec((1, N, 3), lambda b: (b, 0, 0)),
          pl.BlockSpec((1, N, Dh), lambda b: (b, 0, 0)),
      ],
      out_shape=[
          jax.ShapeDtypeStruct((B, N, 3), jnp.float32),
          jax.ShapeDtypeStruct((B, N, Dh), jnp.float32),
      ],
  )(x, h, acc, xacc, node_mask, wh1h, wh1e, bh1, wh2, bh2)


def kernel(x, h, edge_attr, edge_indices, node_mask, edge_mask,
           We1, be1, We2, be2, Wa, ba, Wh1, bh1, Wh2, bh2,
           Wx1, bx1, Wx2, bx2, Wx3):
  B, N, Dh = h.shape
  E = edge_attr.shape[1]
  EBLK = 2048

  idx_i = edge_indices[..., 0].astype(jnp.int32)
  idx_j = edge_indices[..., 1].astype(jnp.int32)

  ta, tb = _make_tables(h, Wx1[0:256], Wx1[256:512], We1[0:256], We1[256:512])

  H = B // 2
  eaccs, xaccs = [], []
  gs = [_sc_gather(ta[s], tb[s], idx_i[s], idx_j[s])
        for s in (slice(0, H), slice(H, B))]
  for (ga, gb), s in zip(gs, (slice(0, H), slice(H, B))):
    ii3 = idx_i[s].reshape(H * (E // EBLK), 1, EBLK)
    ij3 = idx_j[s].reshape(H * (E // EBLK), 1, EBLK)
    eacc, xacc = _edge_mlp(
        ga, gb, edge_attr[s], edge_mask[s], x[s], ii3, ij3,
        Wx1[513:529], Wx1[512:513], bx1.reshape(1, 256),
        Wx2, bx2.reshape(1, 256), Wx3.reshape(1, 256),
        We1[513:529], We1[512:513], be1.reshape(1, 256),
        We2, be2.reshape(1, 256), Wa.reshape(1, 256), ba.reshape(1, 1))
    eaccs.append(eacc)
    xaccs.append(xacc)
  eacc = jnp.concatenate(eaccs, 0)
  xacc = jnp.concatenate(xaccs, 0)
  return _node_update(x, h, eacc, xacc, node_mask,
                      Wh1[0:Dh], Wh1[Dh:Dh + 256], bh1.reshape(1, 256),
                      Wh2, bh2.reshape(1, Dh))
